# Initial kernel scaffold; baseline (speedup 1.0000x reference)
import jax
import jax.numpy as jnp
from jax.experimental import pallas as pl


def kernel(x, edge_index, edge_attr, triplets, global_feat, batch, params):
    raise NotImplementedError("write your pallas kernel here")


# speedup vs baseline: 1.8679x; 1.8679x over previous
import functools

import jax
import jax.numpy as jnp
from jax import lax
from jax.experimental import pallas as pl

_INTERPRET = False

N = 10000
E = 320000
T = 500000
NH = 128
EDIM = 64
NG = 64
NL = 3

_BR = 2000


def _ln(xv, g, b):
    m = jnp.mean(xv, axis=-1, keepdims=True)
    v = jnp.mean((xv - m) ** 2, axis=-1, keepdims=True)
    return (xv - m) / jnp.sqrt(v + 1e-5) * g + b


def _silu(xv):
    return xv * jax.nn.sigmoid(xv)


def _rows_call(body, nrow_args, out_dim, *args, br=_BR):
    R = args[0].shape[0]
    assert R % br == 0, (R, br)
    grid = (R // br,)

    def _rowspec(a):
        nd = a.ndim
        return pl.BlockSpec((br,) + a.shape[1:], lambda i, _nd=nd: (i,) + (0,) * (_nd - 1))

    def _fullspec(a):
        nd = a.ndim
        return pl.BlockSpec(a.shape, lambda i, _nd=nd: (0,) * _nd)

    in_specs = [_rowspec(a) for a in args[:nrow_args]] + [_fullspec(a) for a in args[nrow_args:]]
    return pl.pallas_call(
        body,
        grid=grid,
        in_specs=in_specs,
        out_specs=pl.BlockSpec((br, out_dim), lambda i: (i, 0)),
        out_shape=jax.ShapeDtypeStruct((R, out_dim), jnp.float32),
        interpret=_INTERPRET,
    )(*args)


def _two_layer_body(x_ref, w1_ref, b1_ref, g1_ref, be1_ref,
                    w2_ref, b2_ref, g2_ref, be2_ref, o_ref):
    a = jnp.dot(x_ref[...], w1_ref[...], preferred_element_type=jnp.float32) + b1_ref[...]
    a = _ln(_silu(a), g1_ref[...], be1_ref[...])
    a = jnp.dot(a, w2_ref[...], preferred_element_type=jnp.float32) + b2_ref[...]
    o_ref[...] = _ln(_silu(a), g2_ref[...], be2_ref[...])


def _node_emb_body(x_ref, w_ref, b_ref, g_ref, be_ref, o_ref):
    a = jnp.dot(x_ref[...], w_ref[...], preferred_element_type=jnp.float32) + b_ref[...]
    o_ref[...] = _silu(_ln(a, g_ref[...], be_ref[...]))


def _triplet_body(gij_ref, gkj_ref, geo_ref,
                  w1a_ref, w1b_ref, w1g_ref, b1_ref, g1_ref, be1_ref,
                  w2_ref, b2_ref, g2_ref, be2_ref, o_ref):
    pre = (jnp.dot(gij_ref[...], w1a_ref[...], preferred_element_type=jnp.float32)
           + jnp.dot(gkj_ref[...], w1b_ref[...], preferred_element_type=jnp.float32)
           + jnp.dot(geo_ref[...], w1g_ref[...], preferred_element_type=jnp.float32)
           + b1_ref[...])
    a = _ln(_silu(pre), g1_ref[...], be1_ref[...])
    a = jnp.dot(a, w2_ref[...], preferred_element_type=jnp.float32) + b2_ref[...]
    o_ref[...] = _ln(_silu(a), g2_ref[...], be2_ref[...])


def _msg_body(hsrc_ref, e_ref,
              w1a_ref, w1b_ref, b1_ref, g1_ref, be1_ref,
              w2_ref, b2_ref, g2_ref, be2_ref, o_ref):
    pre = (jnp.dot(hsrc_ref[...], w1a_ref[...], preferred_element_type=jnp.float32)
           + jnp.dot(e_ref[...], w1b_ref[...], preferred_element_type=jnp.float32)
           + b1_ref[...])
    a = _ln(_silu(pre), g1_ref[...], be1_ref[...])
    a = jnp.dot(a, w2_ref[...], preferred_element_type=jnp.float32) + b2_ref[...]
    o_ref[...] = _ln(_silu(a), g2_ref[...], be2_ref[...])


def _edge_update_body(e_ref, agg_ref, cnt_ref, wpsi_ref, bpsi_ref, g_ref, be_ref, o_ref):
    agg = agg_ref[...] / jnp.maximum(cnt_ref[...], 1.0)
    a = e_ref[...] + jnp.dot(agg, wpsi_ref[...], preferred_element_type=jnp.float32) + bpsi_ref[...]
    o_ref[...] = _ln(a, g_ref[...], be_ref[...])


def _node_update_body(h_ref, agg_ref, cnt_ref, wua_ref, wub_ref, bu_ref, g_ref, be_ref, o_ref):
    magg = agg_ref[...] / jnp.maximum(cnt_ref[...], 1.0)
    h = h_ref[...]
    a = (h + jnp.dot(h, wua_ref[...], preferred_element_type=jnp.float32)
         + jnp.dot(magg, wub_ref[...], preferred_element_type=jnp.float32) + bu_ref[...])
    o_ref[...] = _ln(a, g_ref[...], be_ref[...])


def _pool_body(h_ref, bf_ref, pool_ref, cnt_ref):
    @pl.when(pl.program_id(0) == 0)
    def _():
        pool_ref[...] = jnp.zeros_like(pool_ref)
        cnt_ref[...] = jnp.zeros_like(cnt_ref)
    gids = lax.broadcasted_iota(jnp.int32, (1, NG), 1).astype(jnp.float32)
    onehot = (bf_ref[...] == gids).astype(jnp.float32)
    pool_ref[...] += lax.dot_general(onehot, h_ref[...], (((0,), (0,)), ((), ())),
                                     preferred_element_type=jnp.float32)
    cnt_ref[...] += jnp.sum(onehot, axis=0)[:, None]


def _cls_body(hp_ref, cnt_ref, gf_ref,
              l1a_ref, l1b_ref, b1_ref, g1_ref, be1_ref,
              l2_ref, b2_ref, g2_ref, be2_ref,
              l3_ref, b3_ref, o_ref):
    hp = hp_ref[...] / jnp.maximum(cnt_ref[...], 1.0)
    pre = (jnp.dot(hp, l1a_ref[...], preferred_element_type=jnp.float32)
           + jnp.dot(gf_ref[...], l1b_ref[...], preferred_element_type=jnp.float32)
           + b1_ref[...])
    z = _silu(_ln(pre, g1_ref[...], be1_ref[...]))
    z = jnp.dot(z, l2_ref[...], preferred_element_type=jnp.float32) + b2_ref[...]
    z = _silu(_ln(z, g2_ref[...], be2_ref[...]))
    o_ref[...] = jnp.dot(z, l3_ref[...], preferred_element_type=jnp.float32) + b3_ref[...]


def _gather_rows(table, idx):
    return jnp.take(table, idx, axis=0)


def _seg_sum(vals, ids, n):
    return jax.ops.segment_sum(vals, ids, num_segments=n)


def kernel(x, edge_index, edge_attr, triplets, global_feat, batch, params):
    e_ij = triplets[:, 0].astype(jnp.int32)
    e_kj = triplets[:, 1].astype(jnp.int32)
    geo = jnp.pad(triplets[:, 2:6], ((0, 0), (0, 4)))
    src = edge_index[0]
    dst = edge_index[1]

    cntE = _seg_sum(jnp.ones((T,), jnp.float32), e_ij, E)[:, None]
    cntN = _seg_sum(jnp.ones((E,), jnp.float32), dst, N)[:, None]

    ep = params["edge_init"]
    e = _rows_call(_two_layer_body, 1, EDIM, edge_attr,
                   ep["l1"]["w"], ep["l1"]["b"], ep["n1"]["g"], ep["n1"]["b"],
                   ep["l2"]["w"], ep["l2"]["b"], ep["n2"]["g"], ep["n2"]["b"])

    npm = params["node_emb"]
    h = _rows_call(_node_emb_body, 1, NH, x,
                   npm["lin"]["w"], npm["lin"]["b"], npm["ln"]["g"], npm["ln"]["b"])

    for l in range(NL):
        ap = params["angle"][l]
        gij = _gather_rows(e, e_ij)
        gkj = _gather_rows(e, e_kj)
        w1 = ap["p1"]["w"]
        w1g = jnp.pad(w1[2 * EDIM:], ((0, 4), (0, 0)))
        t = _rows_call(_triplet_body, 3, EDIM, gij, gkj, geo,
                       w1[:EDIM], w1[EDIM:2 * EDIM], w1g, ap["p1"]["b"],
                       ap["pn1"]["g"], ap["pn1"]["b"],
                       ap["p2"]["w"], ap["p2"]["b"], ap["pn2"]["g"], ap["pn2"]["b"])
        aggE = _seg_sum(t, e_ij, E)
        e = _rows_call(_edge_update_body, 3, EDIM, e, aggE, cntE,
                       ap["psi"]["w"], ap["psi"]["b"], ap["norm"]["g"], ap["norm"]["b"])

        nb = params["node"][l]
        hsrc = _gather_rows(h, src)
        we1 = nb["e1"]["w"]
        m = _rows_call(_msg_body, 2, NH, hsrc, e,
                       we1[:NH], we1[NH:], nb["e1"]["b"],
                       nb["en1"]["g"], nb["en1"]["b"],
                       nb["e2"]["w"], nb["e2"]["b"], nb["en2"]["g"], nb["en2"]["b"])
        aggN = _seg_sum(m, dst, N)
        wu = nb["upd"]["w"]
        h = _rows_call(_node_update_body, 3, NH, h, aggN, cntN,
                       wu[:NH], wu[NH:], nb["upd"]["b"], nb["norm"]["g"], nb["norm"]["b"])

    bf = batch.astype(jnp.float32)[:, None]
    pool, cnts = pl.pallas_call(
        _pool_body,
        grid=(N // _BR,),
        in_specs=[pl.BlockSpec((_BR, NH), lambda i: (i, 0)),
                  pl.BlockSpec((_BR, 1), lambda i: (i, 0))],
        out_specs=[pl.BlockSpec((NG, NH), lambda i: (0, 0)),
                   pl.BlockSpec((NG, 1), lambda i: (0, 0))],
        out_shape=[jax.ShapeDtypeStruct((NG, NH), jnp.float32),
                   jax.ShapeDtypeStruct((NG, 1), jnp.float32)],
        interpret=_INTERPRET,
    )(h, bf)

    cp = params["cls"]
    l1 = cp["l1"]["w"]
    out = pl.pallas_call(
        _cls_body,
        out_shape=jax.ShapeDtypeStruct((NG, 10), jnp.float32),
        interpret=_INTERPRET,
    )(pool, cnts, global_feat,
      l1[:NH], l1[NH:], cp["l1"]["b"], cp["n1"]["g"], cp["n1"]["b"],
      cp["l2"]["w"], cp["l2"]["b"], cp["n2"]["g"], cp["n2"]["b"],
      cp["l3"]["w"], cp["l3"]["b"])
    return out


# speedup vs baseline: 2.2865x; 1.2241x over previous
import functools

import jax
import jax.numpy as jnp
from jax import lax
from jax.experimental import pallas as pl
from jax.experimental.pallas import tpu as pltpu
from jax.experimental.pallas import tpu_sc as plsc
from jax.experimental import layout as _layout

_INTERPRET = False

N = 10000
E = 320000
T = 500000
NH = 128
EDIM = 64
NG = 64
NL = 3

_BR = 2000


def _ln(xv, g, b):
    m = jnp.mean(xv, axis=-1, keepdims=True)
    v = jnp.mean((xv - m) ** 2, axis=-1, keepdims=True)
    return (xv - m) / jnp.sqrt(v + 1e-5) * g + b


def _silu(xv):
    return xv * jax.nn.sigmoid(xv)


def _rows_call(body, nrow_args, out_dim, *args, br=_BR):
    R = args[0].shape[0]
    assert R % br == 0, (R, br)
    grid = (R // br,)

    def _rowspec(a):
        nd = a.ndim
        return pl.BlockSpec((br,) + a.shape[1:], lambda i, _nd=nd: (i,) + (0,) * (_nd - 1))

    def _fullspec(a):
        nd = a.ndim
        return pl.BlockSpec(a.shape, lambda i, _nd=nd: (0,) * _nd)

    in_specs = [_rowspec(a) for a in args[:nrow_args]] + [_fullspec(a) for a in args[nrow_args:]]
    return pl.pallas_call(
        body,
        grid=grid,
        in_specs=in_specs,
        out_specs=pl.BlockSpec((br, out_dim), lambda i: (i, 0)),
        out_shape=jax.ShapeDtypeStruct((R, out_dim), jnp.float32),
        interpret=_INTERPRET,
    )(*args)


def _two_layer_body(x_ref, w1_ref, b1_ref, g1_ref, be1_ref,
                    w2_ref, b2_ref, g2_ref, be2_ref, o_ref):
    a = jnp.dot(x_ref[...], w1_ref[...], preferred_element_type=jnp.float32) + b1_ref[...]
    a = _ln(_silu(a), g1_ref[...], be1_ref[...])
    a = jnp.dot(a, w2_ref[...], preferred_element_type=jnp.float32) + b2_ref[...]
    o_ref[...] = _ln(_silu(a), g2_ref[...], be2_ref[...])


def _node_emb_body(x_ref, w_ref, b_ref, g_ref, be_ref, o_ref):
    a = jnp.dot(x_ref[...], w_ref[...], preferred_element_type=jnp.float32) + b_ref[...]
    o_ref[...] = _silu(_ln(a, g_ref[...], be_ref[...]))


def _triplet_body(gij_ref, gkj_ref, geo_ref,
                  w1a_ref, w1b_ref, w1g_ref, b1_ref, g1_ref, be1_ref,
                  w2_ref, b2_ref, g2_ref, be2_ref, o_ref):
    pre = (jnp.dot(gij_ref[...], w1a_ref[...], preferred_element_type=jnp.float32)
           + jnp.dot(gkj_ref[...], w1b_ref[...], preferred_element_type=jnp.float32)
           + jnp.dot(geo_ref[...], w1g_ref[...], preferred_element_type=jnp.float32)
           + b1_ref[...])
    a = _ln(_silu(pre), g1_ref[...], be1_ref[...])
    a = jnp.dot(a, w2_ref[...], preferred_element_type=jnp.float32) + b2_ref[...]
    o_ref[...] = _ln(_silu(a), g2_ref[...], be2_ref[...])


def _msg_body(hsrc_ref, e_ref,
              w1a_ref, w1b_ref, b1_ref, g1_ref, be1_ref,
              w2_ref, b2_ref, g2_ref, be2_ref, o_ref):
    pre = (jnp.dot(hsrc_ref[...], w1a_ref[...], preferred_element_type=jnp.float32)
           + jnp.dot(e_ref[...], w1b_ref[...], preferred_element_type=jnp.float32)
           + b1_ref[...])
    a = _ln(_silu(pre), g1_ref[...], be1_ref[...])
    a = jnp.dot(a, w2_ref[...], preferred_element_type=jnp.float32) + b2_ref[...]
    o_ref[...] = _ln(_silu(a), g2_ref[...], be2_ref[...])


def _edge_update_body(e_ref, agg_ref, cnt_ref, wpsi_ref, bpsi_ref, g_ref, be_ref, o_ref):
    agg = agg_ref[...] / jnp.maximum(cnt_ref[...], 1.0)
    a = e_ref[...] + jnp.dot(agg, wpsi_ref[...], preferred_element_type=jnp.float32) + bpsi_ref[...]
    o_ref[...] = _ln(a, g_ref[...], be_ref[...])


def _node_update_body(h_ref, agg_ref, cnt_ref, wua_ref, wub_ref, bu_ref, g_ref, be_ref, o_ref):
    magg = agg_ref[...] / jnp.maximum(cnt_ref[...], 1.0)
    h = h_ref[...]
    a = (h + jnp.dot(h, wua_ref[...], preferred_element_type=jnp.float32)
         + jnp.dot(magg, wub_ref[...], preferred_element_type=jnp.float32) + bu_ref[...])
    o_ref[...] = _ln(a, g_ref[...], be_ref[...])


def _pool_body(h_ref, bf_ref, pool_ref, cnt_ref):
    @pl.when(pl.program_id(0) == 0)
    def _():
        pool_ref[...] = jnp.zeros_like(pool_ref)
        cnt_ref[...] = jnp.zeros_like(cnt_ref)
    gids = lax.broadcasted_iota(jnp.int32, (1, NG), 1).astype(jnp.float32)
    onehot = (bf_ref[...] == gids).astype(jnp.float32)
    pool_ref[...] += lax.dot_general(onehot, h_ref[...], (((0,), (0,)), ((), ())),
                                     preferred_element_type=jnp.float32)
    cnt_ref[...] += jnp.sum(onehot, axis=0)[:, None]


def _cls_body(hp_ref, cnt_ref, gf_ref,
              l1a_ref, l1b_ref, b1_ref, g1_ref, be1_ref,
              l2_ref, b2_ref, g2_ref, be2_ref,
              l3_ref, b3_ref, o_ref):
    hp = hp_ref[...] / jnp.maximum(cnt_ref[...], 1.0)
    pre = (jnp.dot(hp, l1a_ref[...], preferred_element_type=jnp.float32)
           + jnp.dot(gf_ref[...], l1b_ref[...], preferred_element_type=jnp.float32)
           + b1_ref[...])
    z = _silu(_ln(pre, g1_ref[...], be1_ref[...]))
    z = jnp.dot(z, l2_ref[...], preferred_element_type=jnp.float32) + b2_ref[...]
    z = _silu(_ln(z, g2_ref[...], be2_ref[...]))
    o_ref[...] = jnp.dot(z, l3_ref[...], preferred_element_type=jnp.float32) + b3_ref[...]


_NW = 32


def _sc_gather_call(table, idx_pad, batch, nb):
    kp = idx_pad.shape[0]
    d = table.shape[1]
    table = _layout.with_layout_constraint(
        table, _layout.Layout(major_to_minor=(0, 1), tiling=((8,),)))
    bpw = batch * nb
    assert kp == _NW * bpw and batch % 8 == 0, (kp, batch, nb)
    mesh = plsc.VectorSubcoreMesh(core_axis_name="c", subcore_axis_name="s")

    @functools.partial(
        pl.kernel, mesh=mesh,
        out_type=jax.ShapeDtypeStruct((kp, d), jnp.float32),
        scratch_types=[pltpu.VMEM((batch,), jnp.int32),
                       pltpu.VMEM((batch, d), jnp.float32),
                       pltpu.SemaphoreType.DMA],
    )
    def k(table_hbm, idx_hbm, out_hbm, idx_v, rows_v, sem):
        wid = lax.axis_index("s") * 2 + lax.axis_index("c")

        def body(j, carry):
            base = wid * bpw + j * batch
            pltpu.sync_copy(idx_hbm.at[pl.ds(base, batch)], idx_v)
            pltpu.async_copy(table_hbm.at[idx_v], rows_v, sem).wait()
            pltpu.sync_copy(rows_v, out_hbm.at[pl.ds(base, batch)])
            return carry

        lax.fori_loop(0, nb, body, 0)

    return k(table, idx_pad)


def _pad_idx(idx, kp, nrows):
    p = kp - idx.shape[0]
    if p == 0:
        return idx
    fill = (jnp.arange(p, dtype=jnp.int32) * 64) % nrows
    return jnp.concatenate([idx, fill])


def _seg_sum(vals, ids, n):
    return jax.ops.segment_sum(vals, ids, num_segments=n)


_TBATCH, _TNB = 512, 32
_TP = _NW * _TBATCH * _TNB
_EBATCH, _ENB = 400, 25


def kernel(x, edge_index, edge_attr, triplets, global_feat, batch, params):
    e_ij = triplets[:, 0].astype(jnp.int32)
    e_kj = triplets[:, 1].astype(jnp.int32)
    e_ij_p = _pad_idx(e_ij, _TP, E)
    e_kj_p = _pad_idx(e_kj, _TP, E)
    geo = jnp.pad(triplets[:, 2:6], ((0, _TP - T), (0, 4)))
    src = edge_index[0]
    dst = edge_index[1]

    cntE = _seg_sum(jnp.ones((T,), jnp.float32), e_ij, E)[:, None]
    cntN = _seg_sum(jnp.ones((E,), jnp.float32), dst, N)[:, None]

    ep = params["edge_init"]
    e = _rows_call(_two_layer_body, 1, EDIM, edge_attr,
                   ep["l1"]["w"], ep["l1"]["b"], ep["n1"]["g"], ep["n1"]["b"],
                   ep["l2"]["w"], ep["l2"]["b"], ep["n2"]["g"], ep["n2"]["b"])

    npm = params["node_emb"]
    h = _rows_call(_node_emb_body, 1, NH, x,
                   npm["lin"]["w"], npm["lin"]["b"], npm["ln"]["g"], npm["ln"]["b"])

    for l in range(NL):
        ap = params["angle"][l]
        gij = _sc_gather_call(e, e_ij_p, _TBATCH, _TNB)
        gkj = _sc_gather_call(e, e_kj_p, _TBATCH, _TNB)
        w1 = ap["p1"]["w"]
        w1g = jnp.pad(w1[2 * EDIM:], ((0, 4), (0, 0)))
        t = _rows_call(_triplet_body, 3, EDIM, gij, gkj, geo,
                       w1[:EDIM], w1[EDIM:2 * EDIM], w1g, ap["p1"]["b"],
                       ap["pn1"]["g"], ap["pn1"]["b"],
                       ap["p2"]["w"], ap["p2"]["b"], ap["pn2"]["g"], ap["pn2"]["b"],
                       br=2048)
        aggE = _seg_sum(t[:T], e_ij, E)
        e = _rows_call(_edge_update_body, 3, EDIM, e, aggE, cntE,
                       ap["psi"]["w"], ap["psi"]["b"], ap["norm"]["g"], ap["norm"]["b"])

        nb = params["node"][l]
        hsrc = _sc_gather_call(h, src, _EBATCH, _ENB)
        we1 = nb["e1"]["w"]
        m = _rows_call(_msg_body, 2, NH, hsrc, e,
                       we1[:NH], we1[NH:], nb["e1"]["b"],
                       nb["en1"]["g"], nb["en1"]["b"],
                       nb["e2"]["w"], nb["e2"]["b"], nb["en2"]["g"], nb["en2"]["b"])
        aggN = _seg_sum(m, dst, N)
        wu = nb["upd"]["w"]
        h = _rows_call(_node_update_body, 3, NH, h, aggN, cntN,
                       wu[:NH], wu[NH:], nb["upd"]["b"], nb["norm"]["g"], nb["norm"]["b"])

    bf = batch.astype(jnp.float32)[:, None]
    pool, cnts = pl.pallas_call(
        _pool_body,
        grid=(N // _BR,),
        in_specs=[pl.BlockSpec((_BR, NH), lambda i: (i, 0)),
                  pl.BlockSpec((_BR, 1), lambda i: (i, 0))],
        out_specs=[pl.BlockSpec((NG, NH), lambda i: (0, 0)),
                   pl.BlockSpec((NG, 1), lambda i: (0, 0))],
        out_shape=[jax.ShapeDtypeStruct((NG, NH), jnp.float32),
                   jax.ShapeDtypeStruct((NG, 1), jnp.float32)],
        interpret=_INTERPRET,
    )(h, bf)

    cp = params["cls"]
    l1 = cp["l1"]["w"]
    out = pl.pallas_call(
        _cls_body,
        out_shape=jax.ShapeDtypeStruct((NG, 10), jnp.float32),
        interpret=_INTERPRET,
    )(pool, cnts, global_feat,
      l1[:NH], l1[NH:], cp["l1"]["b"], cp["n1"]["g"], cp["n1"]["b"],
      cp["l2"]["w"], cp["l2"]["b"], cp["n2"]["g"], cp["n2"]["b"],
      cp["l3"]["w"], cp["l3"]["b"])
    return out


# speedup vs baseline: 2.3026x; 1.0070x over previous
import functools

import jax
import jax.numpy as jnp
from jax import lax
from jax.experimental import pallas as pl
from jax.experimental.pallas import tpu as pltpu
from jax.experimental.pallas import tpu_sc as plsc
from jax.experimental import layout as _layout

_INTERPRET = False

N = 10000
E = 320000
T = 500000
NH = 128
EDIM = 64
NG = 64
NL = 3

_BR = 2000


def _ln(xv, g, b):
    m = jnp.mean(xv, axis=-1, keepdims=True)
    v = jnp.mean((xv - m) ** 2, axis=-1, keepdims=True)
    return (xv - m) / jnp.sqrt(v + 1e-5) * g + b


def _silu(xv):
    return xv * jax.nn.sigmoid(xv)


def _rows_call(body, nrow_args, out_dim, *args, br=_BR):
    R = args[0].shape[0]
    assert R % br == 0, (R, br)
    grid = (R // br,)

    def _rowspec(a):
        nd = a.ndim
        return pl.BlockSpec((br,) + a.shape[1:], lambda i, _nd=nd: (i,) + (0,) * (_nd - 1))

    def _fullspec(a):
        nd = a.ndim
        return pl.BlockSpec(a.shape, lambda i, _nd=nd: (0,) * _nd)

    in_specs = [_rowspec(a) for a in args[:nrow_args]] + [_fullspec(a) for a in args[nrow_args:]]
    return pl.pallas_call(
        body,
        grid=grid,
        in_specs=in_specs,
        out_specs=pl.BlockSpec((br, out_dim), lambda i: (i, 0)),
        out_shape=jax.ShapeDtypeStruct((R, out_dim), jnp.float32),
        interpret=_INTERPRET,
    )(*args)


def _two_layer_body(x_ref, w1_ref, b1_ref, g1_ref, be1_ref,
                    w2_ref, b2_ref, g2_ref, be2_ref, o_ref):
    a = jnp.dot(x_ref[...], w1_ref[...], preferred_element_type=jnp.float32) + b1_ref[...]
    a = _ln(_silu(a), g1_ref[...], be1_ref[...])
    a = jnp.dot(a, w2_ref[...], preferred_element_type=jnp.float32) + b2_ref[...]
    o_ref[...] = _ln(_silu(a), g2_ref[...], be2_ref[...])


def _node_emb_body(x_ref, w_ref, b_ref, g_ref, be_ref, o_ref):
    a = jnp.dot(x_ref[...], w_ref[...], preferred_element_type=jnp.float32) + b_ref[...]
    o_ref[...] = _silu(_ln(a, g_ref[...], be_ref[...]))


def _triplet_body(gij_ref, gkj_ref, geo_ref,
                  w1a_ref, w1b_ref, w1g_ref, b1_ref, g1_ref, be1_ref,
                  w2_ref, b2_ref, g2_ref, be2_ref, o_ref):
    pre = (jnp.dot(gij_ref[...], w1a_ref[...], preferred_element_type=jnp.float32)
           + jnp.dot(gkj_ref[...], w1b_ref[...], preferred_element_type=jnp.float32)
           + jnp.dot(geo_ref[...], w1g_ref[...], preferred_element_type=jnp.float32)
           + b1_ref[...])
    a = _ln(_silu(pre), g1_ref[...], be1_ref[...])
    a = jnp.dot(a, w2_ref[...], preferred_element_type=jnp.float32) + b2_ref[...]
    o_ref[...] = _ln(_silu(a), g2_ref[...], be2_ref[...])


def _msg_body(hsrc_ref, e_ref,
              w1a_ref, w1b_ref, b1_ref, g1_ref, be1_ref,
              w2_ref, b2_ref, g2_ref, be2_ref, o_ref):
    pre = (jnp.dot(hsrc_ref[...], w1a_ref[...], preferred_element_type=jnp.float32)
           + jnp.dot(e_ref[...], w1b_ref[...], preferred_element_type=jnp.float32)
           + b1_ref[...])
    a = _ln(_silu(pre), g1_ref[...], be1_ref[...])
    a = jnp.dot(a, w2_ref[...], preferred_element_type=jnp.float32) + b2_ref[...]
    o_ref[...] = _ln(_silu(a), g2_ref[...], be2_ref[...])


def _edge_update_body(e_ref, agg_ref, cnt_ref, wpsi_ref, bpsi_ref, g_ref, be_ref, o_ref):
    agg = agg_ref[...] / jnp.maximum(cnt_ref[...], 1.0)
    a = e_ref[...] + jnp.dot(agg, wpsi_ref[...], preferred_element_type=jnp.float32) + bpsi_ref[...]
    o_ref[...] = _ln(a, g_ref[...], be_ref[...])


def _node_update_body(h_ref, agg_ref, cnt_ref, wua_ref, wub_ref, bu_ref, g_ref, be_ref, o_ref):
    magg = agg_ref[...] / jnp.maximum(cnt_ref[...], 1.0)
    h = h_ref[...]
    a = (h + jnp.dot(h, wua_ref[...], preferred_element_type=jnp.float32)
         + jnp.dot(magg, wub_ref[...], preferred_element_type=jnp.float32) + bu_ref[...])
    o_ref[...] = _ln(a, g_ref[...], be_ref[...])


def _pool_body(h_ref, bf_ref, pool_ref, cnt_ref):
    @pl.when(pl.program_id(0) == 0)
    def _():
        pool_ref[...] = jnp.zeros_like(pool_ref)
        cnt_ref[...] = jnp.zeros_like(cnt_ref)
    gids = lax.broadcasted_iota(jnp.int32, (1, NG), 1).astype(jnp.float32)
    onehot = (bf_ref[...] == gids).astype(jnp.float32)
    pool_ref[...] += lax.dot_general(onehot, h_ref[...], (((0,), (0,)), ((), ())),
                                     preferred_element_type=jnp.float32)
    cnt_ref[...] += jnp.sum(onehot, axis=0)[:, None]


def _cls_body(hp_ref, cnt_ref, gf_ref,
              l1a_ref, l1b_ref, b1_ref, g1_ref, be1_ref,
              l2_ref, b2_ref, g2_ref, be2_ref,
              l3_ref, b3_ref, o_ref):
    hp = hp_ref[...] / jnp.maximum(cnt_ref[...], 1.0)
    pre = (jnp.dot(hp, l1a_ref[...], preferred_element_type=jnp.float32)
           + jnp.dot(gf_ref[...], l1b_ref[...], preferred_element_type=jnp.float32)
           + b1_ref[...])
    z = _silu(_ln(pre, g1_ref[...], be1_ref[...]))
    z = jnp.dot(z, l2_ref[...], preferred_element_type=jnp.float32) + b2_ref[...]
    z = _silu(_ln(z, g2_ref[...], be2_ref[...]))
    o_ref[...] = jnp.dot(z, l3_ref[...], preferred_element_type=jnp.float32) + b3_ref[...]


_NW = 32


def _sc_gather_call(table, idx_pad, batch, nb):
    kp = idx_pad.shape[0]
    d = table.shape[1]
    table = _layout.with_layout_constraint(
        table, _layout.Layout(major_to_minor=(0, 1), tiling=((8,),)))
    bpw = batch * nb
    assert kp == _NW * bpw and batch % 8 == 0, (kp, batch, nb)
    mesh = plsc.VectorSubcoreMesh(core_axis_name="c", subcore_axis_name="s")

    @functools.partial(
        pl.kernel, mesh=mesh,
        out_type=jax.ShapeDtypeStruct((kp, d), jnp.float32),
        scratch_types=[pltpu.VMEM((batch,), jnp.int32),
                       pltpu.VMEM((batch, d), jnp.float32),
                       pltpu.SemaphoreType.DMA],
    )
    def k(table_hbm, idx_hbm, out_hbm, idx_v, rows_v, sem):
        wid = lax.axis_index("s") * 2 + lax.axis_index("c")

        def body(j, carry):
            base = wid * bpw + j * batch
            pltpu.sync_copy(idx_hbm.at[pl.ds(base, batch)], idx_v)
            pltpu.async_copy(table_hbm.at[idx_v], rows_v, sem).wait()
            pltpu.sync_copy(rows_v, out_hbm.at[pl.ds(base, batch)])
            return carry

        lax.fori_loop(0, nb, body, 0)

    return k(table, idx_pad)


def _pad_idx(idx, kp, nrows):
    p = kp - idx.shape[0]
    if p == 0:
        return idx
    fill = (jnp.arange(p, dtype=jnp.int32) * 64) % nrows
    return jnp.concatenate([idx, fill])


def _seg_sum(vals, ids, n):
    return jax.ops.segment_sum(vals, ids, num_segments=n)


_TBATCH, _TNB = 512, 32
_TP = _NW * _TBATCH * _TNB
_EBATCH, _ENB = 400, 25


def kernel(x, edge_index, edge_attr, triplets, global_feat, batch, params):
    e_ij = triplets[:, 0].astype(jnp.int32)
    e_kj = triplets[:, 1].astype(jnp.int32)
    e_ij_p = _pad_idx(e_ij, _TP, E)
    e_kj_p = _pad_idx(e_kj, _TP, E)
    e_ij_seg = jnp.concatenate([e_ij, jnp.full((_TP - T,), E, jnp.int32)])
    geo = jnp.pad(triplets[:, 2:6], ((0, _TP - T), (0, 4)))
    src = edge_index[0]
    dst = edge_index[1]

    cntE = _seg_sum(jnp.ones((T,), jnp.float32), e_ij, E)[:, None]
    cntN = _seg_sum(jnp.ones((E,), jnp.float32), dst, N)[:, None]

    ep = params["edge_init"]
    e = _rows_call(_two_layer_body, 1, EDIM, edge_attr,
                   ep["l1"]["w"], ep["l1"]["b"], ep["n1"]["g"], ep["n1"]["b"],
                   ep["l2"]["w"], ep["l2"]["b"], ep["n2"]["g"], ep["n2"]["b"])

    npm = params["node_emb"]
    h = _rows_call(_node_emb_body, 1, NH, x,
                   npm["lin"]["w"], npm["lin"]["b"], npm["ln"]["g"], npm["ln"]["b"])

    for l in range(NL):
        ap = params["angle"][l]
        gij = _sc_gather_call(e, e_ij_p, _TBATCH, _TNB)
        gkj = _sc_gather_call(e, e_kj_p, _TBATCH, _TNB)
        w1 = ap["p1"]["w"]
        w1g = jnp.pad(w1[2 * EDIM:], ((0, 4), (0, 0)))
        t = _rows_call(_triplet_body, 3, EDIM, gij, gkj, geo,
                       w1[:EDIM], w1[EDIM:2 * EDIM], w1g, ap["p1"]["b"],
                       ap["pn1"]["g"], ap["pn1"]["b"],
                       ap["p2"]["w"], ap["p2"]["b"], ap["pn2"]["g"], ap["pn2"]["b"],
                       br=2048)
        aggE = _seg_sum(t, e_ij_seg, E)
        e = _rows_call(_edge_update_body, 3, EDIM, e, aggE, cntE,
                       ap["psi"]["w"], ap["psi"]["b"], ap["norm"]["g"], ap["norm"]["b"])

        nb = params["node"][l]
        hsrc = _sc_gather_call(h, src, _EBATCH, _ENB)
        we1 = nb["e1"]["w"]
        m = _rows_call(_msg_body, 2, NH, hsrc, e,
                       we1[:NH], we1[NH:], nb["e1"]["b"],
                       nb["en1"]["g"], nb["en1"]["b"],
                       nb["e2"]["w"], nb["e2"]["b"], nb["en2"]["g"], nb["en2"]["b"])
        aggN = _seg_sum(m, dst, N)
        wu = nb["upd"]["w"]
        h = _rows_call(_node_update_body, 3, NH, h, aggN, cntN,
                       wu[:NH], wu[NH:], nb["upd"]["b"], nb["norm"]["g"], nb["norm"]["b"])

    bf = batch.astype(jnp.float32)[:, None]
    pool, cnts = pl.pallas_call(
        _pool_body,
        grid=(N // _BR,),
        in_specs=[pl.BlockSpec((_BR, NH), lambda i: (i, 0)),
                  pl.BlockSpec((_BR, 1), lambda i: (i, 0))],
        out_specs=[pl.BlockSpec((NG, NH), lambda i: (0, 0)),
                   pl.BlockSpec((NG, 1), lambda i: (0, 0))],
        out_shape=[jax.ShapeDtypeStruct((NG, NH), jnp.float32),
                   jax.ShapeDtypeStruct((NG, 1), jnp.float32)],
        interpret=_INTERPRET,
    )(h, bf)

    cp = params["cls"]
    l1 = cp["l1"]["w"]
    out = pl.pallas_call(
        _cls_body,
        out_shape=jax.ShapeDtypeStruct((NG, 10), jnp.float32),
        interpret=_INTERPRET,
    )(pool, cnts, global_feat,
      l1[:NH], l1[NH:], cp["l1"]["b"], cp["n1"]["g"], cp["n1"]["b"],
      cp["l2"]["w"], cp["l2"]["b"], cp["n2"]["g"], cp["n2"]["b"],
      cp["l3"]["w"], cp["l3"]["b"])
    return out
